# Initial kernel scaffold; baseline (speedup 1.0000x reference)
import jax
import jax.numpy as jnp
from jax.experimental import pallas as pl


def kernel(edges, W):
    raise NotImplementedError("write your pallas kernel here")


# speedup vs baseline: 242.3916x; 242.3916x over previous
import functools

import jax
import jax.numpy as jnp
from jax import lax
from jax.experimental import pallas as pl
from jax.experimental.pallas import tpu as pltpu
from jax.experimental.pallas import tpu_sc as plsc

_L = 16


def _make_sc_kernel(E, n_nodes, NC, NS, C):
    NW = NC * NS
    per_w = E // NW
    n_chunks = per_w // C
    mesh = plsc.VectorSubcoreMesh(core_axis_name="c", subcore_axis_name="s")

    @functools.partial(
        pl.kernel,
        out_type=jax.ShapeDtypeStruct((E,), jnp.float32),
        mesh=mesh,
        scratch_types=[
            pltpu.VMEM((C,), jnp.int32),
            pltpu.VMEM((C,), jnp.int32),
            pltpu.VMEM((C,), jnp.float32),
            pltpu.VMEM((C,), jnp.float32),
            pltpu.VMEM((C,), jnp.float32),
            pltpu.VMEM((n_nodes // 10,), jnp.float32),
            pltpu.VMEM_SHARED((n_nodes,), jnp.float32),
            pltpu.VMEM_SHARED((n_nodes,), jnp.float32),
            pltpu.SemaphoreType.DMA,
            pltpu.SemaphoreType.DMA,
        ],
    )
    def k(src_hbm, dst_hbm, w_hbm, out_hbm, si_v, di_v, sv_v, dv_v, out_v,
          tmp_v, ws_sh, wd_sh, sem0, sem1):
        cid = lax.axis_index("c")
        sid = lax.axis_index("s")
        wid = sid * NC + cid

        piece = n_nodes // 10
        for p in range(20):
            half = p // 10
            off = (p % 10) * piece
            sh = ws_sh if half == 0 else wd_sh

            @pl.when(sid == (p % 16))
            def _(off=off, sh=sh, half=half):
                pltpu.sync_copy(w_hbm.at[pl.ds(half * n_nodes + off, piece)],
                                tmp_v)
                pltpu.sync_copy(tmp_v, sh.at[pl.ds(off, piece)])

        plsc.subcore_barrier()

        def chunk_body(j, carry):
            base = wid * per_w + j * C
            pltpu.sync_copy(src_hbm.at[pl.ds(base, C)], si_v)
            pltpu.sync_copy(dst_hbm.at[pl.ds(base, C)], di_v)
            cs = pltpu.async_copy(ws_sh.at[si_v], sv_v, sem0)
            cd = pltpu.async_copy(wd_sh.at[di_v], dv_v, sem1)
            cs.wait()
            cd.wait()

            def comb(t, c):
                s = sv_v[pl.ds(t * _L, _L)] + dv_v[pl.ds(t * _L, _L)]
                out_v[pl.ds(t * _L, _L)] = 1.0 / (1.0 + jnp.exp(-s))
                return c

            lax.fori_loop(0, C // _L, comb, 0, unroll=8)

            pltpu.sync_copy(out_v, out_hbm.at[pl.ds(base, C)])
            return carry

        lax.fori_loop(0, n_chunks, chunk_body, 0)

    return k


def kernel(edges, W):
    E = edges.shape[0]
    n_nodes = W.shape[1] // 2
    info = plsc.get_sparse_core_info()
    NC, NS = info.num_cores, info.num_subcores
    C = 8000
    assert E % (NC * NS * C) == 0
    src = edges[:, 0]
    dst = edges[:, 1]
    w = W.reshape(-1)
    return _make_sc_kernel(E, n_nodes, NC, NS, C)(src, dst, w)


# speedup vs baseline: 326.5797x; 1.3473x over previous
import functools

import jax
import jax.numpy as jnp
from jax import lax
from jax.experimental import pallas as pl
from jax.experimental.pallas import tpu as pltpu
from jax.experimental.pallas import tpu_sc as plsc

_L = 16


def _make_sc_kernel(E, n_nodes, NC, NS, C):
    NW = NC * NS
    per_w = E // NW
    n_chunks = per_w // C
    n_pieces = 10
    piece = n_nodes // n_pieces
    mesh = plsc.VectorSubcoreMesh(core_axis_name="c", subcore_axis_name="s")

    @functools.partial(
        pl.kernel,
        out_type=jax.ShapeDtypeStruct((E,), jnp.float32),
        mesh=mesh,
        scratch_types=[
            [pltpu.VMEM((C,), jnp.int32) for _ in range(2)],
            [pltpu.VMEM((C,), jnp.int32) for _ in range(2)],
            [pltpu.VMEM((C,), jnp.float32) for _ in range(2)],
            [pltpu.VMEM((C,), jnp.float32) for _ in range(2)],
            [pltpu.VMEM((C,), jnp.float32) for _ in range(2)],
            pltpu.VMEM_SHARED((n_nodes,), jnp.float32),
            pltpu.VMEM_SHARED((n_nodes,), jnp.float32),
            [pltpu.SemaphoreType.DMA for _ in range(2)],
            [pltpu.SemaphoreType.DMA for _ in range(2)],
            [pltpu.SemaphoreType.DMA for _ in range(2)],
        ],
    )
    def k(src_hbm, dst_hbm, w_hbm, out_hbm, si, di, sv, dv, ov,
          ws_sh, wd_sh, isem, gsem, osem):
        cid = lax.axis_index("c")
        sid = lax.axis_index("s")
        wid = sid * NC + cid
        base0 = wid * per_w

        for p in range(2 * n_pieces):
            half = p // n_pieces
            off = (p % n_pieces) * piece
            sh = ws_sh if half == 0 else wd_sh
            tmp = sv[p % 2]

            @pl.when(sid == (p % NS))
            def _(off=off, sh=sh, half=half, tmp=tmp):
                pltpu.sync_copy(w_hbm.at[pl.ds(half * n_nodes + off, piece)],
                                tmp.at[pl.ds(0, piece)])
                pltpu.sync_copy(tmp.at[pl.ds(0, piece)],
                                sh.at[pl.ds(off, piece)])

        plsc.subcore_barrier()

        def start_idx(j):
            b = j % 2
            ci = pltpu.async_copy(src_hbm.at[pl.ds(base0 + j * C, C)],
                                  si[b], isem[b])
            cj = pltpu.async_copy(dst_hbm.at[pl.ds(base0 + j * C, C)],
                                  di[b], isem[b])
            return (ci, cj)

        def start_gather(j):
            b = j % 2
            cs = pltpu.async_copy(ws_sh.at[si[b]], sv[b], gsem[b])
            cd = pltpu.async_copy(wd_sh.at[di[b]], dv[b], gsem[b])
            return (cs, cd)

        idx_d = {0: start_idx(0)}
        if n_chunks > 1:
            idx_d[1] = start_idx(1)
        for c in idx_d[0]:
            c.wait()
        g_d = {0: start_gather(0)}
        o_d = {}

        for j in range(n_chunks):
            b = j % 2
            for c in g_d.pop(j):
                c.wait()
            if j + 1 < n_chunks:
                for c in idx_d.pop(j + 1):
                    c.wait()
                g_d[j + 1] = start_gather(j + 1)
            if j >= 2:
                o_d.pop(j - 2).wait()

            svb, dvb, ovb = sv[b], dv[b], ov[b]

            def comb(t, carry, svb=svb, dvb=dvb, ovb=ovb):
                s = svb[pl.ds(t * _L, _L)] + dvb[pl.ds(t * _L, _L)]
                ovb[pl.ds(t * _L, _L)] = 1.0 / (1.0 + jnp.exp(-s))
                return carry

            lax.fori_loop(0, C // _L, comb, 0, unroll=8)

            o_d[j] = pltpu.async_copy(ov[b], out_hbm.at[pl.ds(base0 + j * C, C)],
                                      osem[b])
            if j + 2 < n_chunks:
                idx_d[j + 2] = start_idx(j + 2)

        for j, c in sorted(o_d.items()):
            c.wait()

    return k


def kernel(edges, W):
    E = edges.shape[0]
    n_nodes = W.shape[1] // 2
    info = plsc.get_sparse_core_info()
    NC, NS = info.num_cores, info.num_subcores
    C = 10000
    assert E % (NC * NS * C) == 0
    src = edges[:, 0]
    dst = edges[:, 1]
    w = W.reshape(-1)
    return _make_sc_kernel(E, n_nodes, NC, NS, C)(src, dst, w)
